# Initial kernel scaffold; baseline (speedup 1.0000x reference)
import jax
import jax.numpy as jnp
from jax.experimental import pallas as pl


def kernel(x, edge_index, edge_weight, weights):
    raise NotImplementedError("write your pallas kernel here")


# speedup vs baseline: 3.4482x; 3.4482x over previous
import functools

import jax
import jax.numpy as jnp
from jax import lax
from jax.experimental import pallas as pl
from jax.experimental.pallas import tpu as pltpu
from jax.experimental.pallas import tpu_sc as plsc

NC = 2
NS = 16
NW = NC * NS
LANES = 16
CHUNK = 80


def _make_spmm(n, e, f):
    epw = e // NW
    nchunk = epw // CHUNK
    n_pad = -(-n // 128) * 128
    rpw = n_pad // NS
    mesh = plsc.VectorSubcoreMesh(core_axis_name="c", subcore_axis_name="s")

    @functools.partial(
        pl.kernel,
        out_type=jax.ShapeDtypeStruct((2 * n_pad, f), jnp.float32),
        mesh=mesh,
        compiler_params=pltpu.CompilerParams(needs_layout_passes=False),
        scratch_types=[
            pltpu.VMEM((CHUNK,), jnp.int32),
            pltpu.VMEM((CHUNK,), jnp.int32),
            pltpu.VMEM((CHUNK,), jnp.float32),
            pltpu.VMEM((CHUNK, f), jnp.float32),
            pltpu.VMEM_SHARED((n_pad, f), jnp.float32),
            pltpu.SemaphoreType.DMA,
        ],
    )
    def spmm(table, cols_h, rows_h, vals_h, zeros_h, out, colv, rowv, valv,
             g, acc, sem):
        c = lax.axis_index("c")
        s = lax.axis_index("s")
        wid = s * NC + c

        pltpu.sync_copy(zeros_h.at[pl.ds(s * rpw, rpw)],
                        acc.at[pl.ds(s * rpw, rpw)])
        plsc.subcore_barrier()

        base = wid * epw
        iota = lax.iota(jnp.int32, LANES)

        def chunk_body(i, carry):
            off = base + i * CHUNK
            pltpu.sync_copy(cols_h.at[pl.ds(off, CHUNK)], colv)
            pltpu.sync_copy(rows_h.at[pl.ds(off, CHUNK)], rowv)
            pltpu.sync_copy(vals_h.at[pl.ds(off, CHUNK)], valv)
            pltpu.async_copy(table.at[colv], g, sem).wait()

            def grp_body(grp, carry2):
                for k in range(LANES):
                    eidx = grp * LANES + k
                    evec = jnp.broadcast_to(eidx, (LANES,))
                    vb = plsc.load_gather(valv, [evec])
                    for j in range(f // LANES):
                        sl = pl.ds(j * LANES, LANES)
                        g[eidx, sl] = g[eidx, sl] * vb
                return carry2

            lax.fori_loop(0, CHUNK // LANES, grp_body, 0)

            pltpu.sync_copy(g, acc.at[rowv], add=True)
            return carry

        lax.fori_loop(0, nchunk, chunk_body, 0)
        plsc.subcore_barrier()

        pltpu.sync_copy(acc.at[pl.ds(s * rpw, rpw)],
                        out.at[pl.ds(c * n_pad + s * rpw, rpw)])

    return spmm, n_pad


def _tc1(x, p0, p1, w0, w1, inv_n):
    n, f = x.shape
    blk = 1000

    def body(xr, p0r, p1r, w0r, w1r, z1r, y01r):
        z1 = (p0r[...] + p1r[...]) * inv_n
        z1r[...] = z1
        y01r[...] = (jnp.dot(xr[...], w0r[...],
                             preferred_element_type=jnp.float32)
                     + jnp.dot(z1, w1r[...],
                               preferred_element_type=jnp.float32))

    row_spec = pl.BlockSpec((blk, f), lambda i: (i, 0))
    w_spec = pl.BlockSpec((f, f), lambda i: (0, 0))
    return pl.pallas_call(
        body,
        grid=(n // blk,),
        in_specs=[row_spec, row_spec, row_spec, w_spec, w_spec],
        out_specs=[row_spec, row_spec],
        out_shape=[jax.ShapeDtypeStruct((n, f), jnp.float32),
                   jax.ShapeDtypeStruct((n, f), jnp.float32)],
    )(x, p0, p1, w0, w1)


def _tc2(y01, q0, q1, w2, inv_n):
    n, f = y01.shape
    blk = 1000

    def body(y01r, q0r, q1r, w2r, yr):
        z2 = (q0r[...] + q1r[...]) * inv_n
        yr[...] = y01r[...] + jnp.dot(z2, w2r[...],
                                      preferred_element_type=jnp.float32)

    row_spec = pl.BlockSpec((blk, f), lambda i: (i, 0))
    w_spec = pl.BlockSpec((f, f), lambda i: (0, 0))
    return pl.pallas_call(
        body,
        grid=(n // blk,),
        in_specs=[row_spec, row_spec, row_spec, w_spec],
        out_specs=row_spec,
        out_shape=jax.ShapeDtypeStruct((n, f), jnp.float32),
    )(y01, q0, q1, w2)


def kernel(x, edge_index, edge_weight, weights):
    n, f = x.shape
    e = edge_weight.shape[0]
    rows = edge_index[0]
    cols = edge_index[1]
    inv_n = float(1.0 / n)

    spmm, n_pad = _make_spmm(n, e, f)
    zeros = jnp.zeros((n_pad, f), jnp.float32)
    p = spmm(x, cols, rows, edge_weight, zeros)
    z1, y01 = _tc1(x, p[:n], p[n_pad:n_pad + n], weights[0], weights[1],
                   inv_n)
    q = spmm(z1, cols, rows, edge_weight, zeros)
    return _tc2(y01, q[:n], q[n_pad:n_pad + n], weights[2], inv_n)
